# baseline (speedup 1.0000x reference)
import functools

import jax
import jax.numpy as jnp
from jax.experimental import pallas as pl

B, T, D_IN, D_SAE, K_PER_T = 512, 4, 768, 8192, 32
K = K_PER_T * T

BS_ENC = 1024
BS_DEC = 1024
MB = 64


def _enc_kernel(x_ref, w_ref, b_ref, out_ref):
    acc = b_ref[...]
    for t in range(T):
        acc = acc + jnp.dot(
            x_ref[:, t, :], w_ref[t], preferred_element_type=jnp.float32
        )
    out_ref[...] = acc


def _key_of(pre):
    ubits = jax.lax.bitcast_convert_type(pre, jnp.uint32)
    return jnp.where(
        ubits >= jnp.uint32(0x80000000),
        ~ubits,
        ubits | jnp.uint32(0x80000000),
    )


def _mask_kernel(pre_ref, z_ref):
    pre = pre_ref[...]
    key = _key_of(pre)
    rows = pre.shape[0]

    def cond(c):
        bit, t, cur = c
        return (bit >= 0) & jnp.logical_not(jnp.all(cur == K))

    def body(c):
        bit, t, cur = c
        cand = t | (jnp.uint32(1) << bit.astype(jnp.uint32))
        cnt = jnp.sum((key >= cand).astype(jnp.int32), axis=1, keepdims=True)
        took = cnt >= K
        t = jnp.where(took, cand, t)
        cur = jnp.where(took, cnt, cur)
        return bit - 1, t, cur

    _, t, _ = jax.lax.while_loop(
        cond, body,
        (jnp.int32(31), jnp.zeros((rows, 1), jnp.uint32),
         jnp.full((rows, 1), D_SAE, jnp.int32)),
    )
    z_ref[...] = jnp.where((key >= t) & (pre > 0.0), pre, 0.0)


def _dec_kernel(z_ref, w_ref, x_ref, bd_ref, xhat_ref, loss_ref, *, n_k):
    k = pl.program_id(0)
    w = w_ref[...].reshape(w_ref.shape[0], T * D_IN)
    d = jnp.dot(z_ref[...], w, preferred_element_type=jnp.float32)

    @pl.when(k == 0)
    def _init():
        xhat_ref[...] = d + bd_ref[...].reshape(1, T * D_IN)

    @pl.when(k != 0)
    def _acc():
        xhat_ref[...] += d

    @pl.when(k == n_k - 1)
    def _loss():
        diff = xhat_ref[...] - x_ref[...].reshape(B, T * D_IN)
        loss_ref[...] = jnp.broadcast_to(jnp.sum(diff * diff), loss_ref.shape)


def _run(x, W_enc, W_dec, b_enc, b_dec, interpret=False):
    be2 = b_enc.reshape(1, D_SAE)

    n_s = D_SAE // BS_ENC
    pre = pl.pallas_call(
        _enc_kernel,
        grid=(n_s,),
        in_specs=[
            pl.BlockSpec((B, T, D_IN), lambda j: (0, 0, 0)),
            pl.BlockSpec((T, D_IN, BS_ENC), lambda j: (0, 0, j)),
            pl.BlockSpec((1, BS_ENC), lambda j: (0, j)),
        ],
        out_specs=pl.BlockSpec((B, BS_ENC), lambda j: (0, j)),
        out_shape=jax.ShapeDtypeStruct((B, D_SAE), jnp.float32),
        interpret=interpret,
    )(x, W_enc, be2)

    z = pl.pallas_call(
        _mask_kernel,
        grid=(B // MB,),
        in_specs=[pl.BlockSpec((MB, D_SAE), lambda i: (i, 0))],
        out_specs=pl.BlockSpec((MB, D_SAE), lambda i: (i, 0)),
        out_shape=jax.ShapeDtypeStruct((B, D_SAE), jnp.float32),
        interpret=interpret,
    )(pre)

    n_k = D_SAE // BS_DEC
    D = T * D_IN
    xhat2, loss_parts = pl.pallas_call(
        functools.partial(_dec_kernel, n_k=n_k),
        grid=(n_k,),
        in_specs=[
            pl.BlockSpec((B, BS_DEC), lambda k: (0, k)),
            pl.BlockSpec((BS_DEC, T, D_IN), lambda k: (k, 0, 0)),
            pl.BlockSpec((B, T, D_IN), lambda k: (0, 0, 0)),
            pl.BlockSpec((T, D_IN), lambda k: (0, 0)),
        ],
        out_specs=[
            pl.BlockSpec((B, D), lambda k: (0, 0)),
            pl.BlockSpec((8, 128), lambda k: (0, 0)),
        ],
        out_shape=[
            jax.ShapeDtypeStruct((B, D), jnp.float32),
            jax.ShapeDtypeStruct((8, 128), jnp.float32),
        ],
        interpret=interpret,
    )(z, W_dec, x, b_dec)

    recon_loss = loss_parts[0, 0] / jnp.float32(B * T)
    x_hat = xhat2.reshape(B, T, D_IN)
    return (recon_loss, x_hat, z)


def kernel(x, W_enc, W_dec, b_enc, b_dec):
    return _run(x, W_enc, W_dec, b_enc, b_dec)


# speedup vs baseline: 1.0634x; 1.0634x over previous
import functools

import jax
import jax.numpy as jnp
from jax.experimental import pallas as pl

B, T, D_IN, D_SAE, K_PER_T = 512, 4, 768, 8192, 32
K = K_PER_T * T

BS_ENC = 1024
BS_DEC = 1024
MB = 128


def _enc_kernel(x_ref, w_ref, b_ref, out_ref):
    acc = b_ref[...]
    for t in range(T):
        acc = acc + jnp.dot(
            x_ref[:, t, :], w_ref[t], preferred_element_type=jnp.float32
        )
    out_ref[...] = acc


def _key_of(pre):
    ubits = jax.lax.bitcast_convert_type(pre, jnp.uint32)
    return jnp.where(
        ubits >= jnp.uint32(0x80000000),
        ~ubits,
        ubits | jnp.uint32(0x80000000),
    )


def _mask_kernel(pre_ref, z_ref):
    pre = pre_ref[...]
    key = _key_of(pre)
    rows = pre.shape[0]

    def cond(c):
        bit, t, cur = c
        return (bit >= 0) & jnp.logical_not(jnp.all(cur == K))

    def body(c):
        bit, t, cur = c
        cand = t | (jnp.uint32(1) << bit.astype(jnp.uint32))
        cnt = jnp.sum((key >= cand).astype(jnp.int32), axis=1, keepdims=True)
        took = cnt >= K
        t = jnp.where(took, cand, t)
        cur = jnp.where(took, cnt, cur)
        return bit - 1, t, cur

    _, t, _ = jax.lax.while_loop(
        cond, body,
        (jnp.int32(31), jnp.zeros((rows, 1), jnp.uint32),
         jnp.full((rows, 1), D_SAE, jnp.int32)),
    )
    z_ref[...] = jnp.where((key >= t) & (pre > 0.0), pre, 0.0)


def _dec_kernel(z_ref, w_ref, x_ref, bd_ref, xhat_ref, loss_ref, *, n_k):
    k = pl.program_id(0)
    w = w_ref[...].reshape(w_ref.shape[0], T * D_IN)
    d = jnp.dot(z_ref[...], w, preferred_element_type=jnp.float32)

    @pl.when(k == 0)
    def _init():
        xhat_ref[...] = d + bd_ref[...].reshape(1, T * D_IN)

    @pl.when(k != 0)
    def _acc():
        xhat_ref[...] += d

    @pl.when(k == n_k - 1)
    def _loss():
        diff = xhat_ref[...] - x_ref[...].reshape(B, T * D_IN)
        loss_ref[...] = jnp.broadcast_to(jnp.sum(diff * diff), loss_ref.shape)


def _run(x, W_enc, W_dec, b_enc, b_dec, interpret=False):
    be2 = b_enc.reshape(1, D_SAE)

    n_s = D_SAE // BS_ENC
    pre = pl.pallas_call(
        _enc_kernel,
        grid=(n_s,),
        in_specs=[
            pl.BlockSpec((B, T, D_IN), lambda j: (0, 0, 0)),
            pl.BlockSpec((T, D_IN, BS_ENC), lambda j: (0, 0, j)),
            pl.BlockSpec((1, BS_ENC), lambda j: (0, j)),
        ],
        out_specs=pl.BlockSpec((B, BS_ENC), lambda j: (0, j)),
        out_shape=jax.ShapeDtypeStruct((B, D_SAE), jnp.float32),
        interpret=interpret,
    )(x, W_enc, be2)

    z = pl.pallas_call(
        _mask_kernel,
        grid=(B // MB,),
        in_specs=[pl.BlockSpec((MB, D_SAE), lambda i: (i, 0))],
        out_specs=pl.BlockSpec((MB, D_SAE), lambda i: (i, 0)),
        out_shape=jax.ShapeDtypeStruct((B, D_SAE), jnp.float32),
        interpret=interpret,
    )(pre)

    n_k = D_SAE // BS_DEC
    D = T * D_IN
    xhat2, loss_parts = pl.pallas_call(
        functools.partial(_dec_kernel, n_k=n_k),
        grid=(n_k,),
        in_specs=[
            pl.BlockSpec((B, BS_DEC), lambda k: (0, k)),
            pl.BlockSpec((BS_DEC, T, D_IN), lambda k: (k, 0, 0)),
            pl.BlockSpec((B, T, D_IN), lambda k: (0, 0, 0)),
            pl.BlockSpec((T, D_IN), lambda k: (0, 0)),
        ],
        out_specs=[
            pl.BlockSpec((B, D), lambda k: (0, 0)),
            pl.BlockSpec((8, 128), lambda k: (0, 0)),
        ],
        out_shape=[
            jax.ShapeDtypeStruct((B, D), jnp.float32),
            jax.ShapeDtypeStruct((8, 128), jnp.float32),
        ],
        interpret=interpret,
    )(z, W_dec, x, b_dec)

    recon_loss = loss_parts[0, 0] / jnp.float32(B * T)
    x_hat = xhat2.reshape(B, T, D_IN)
    return (recon_loss, x_hat, z)


def kernel(x, W_enc, W_dec, b_enc, b_dec):
    return _run(x, W_enc, W_dec, b_enc, b_dec)
